# baseline (speedup 1.0000x reference)
import functools

import jax
import jax.numpy as jnp
from jax import lax
from jax.experimental import pallas as pl
from jax.experimental.pallas import tpu as pltpu
from jax.experimental.pallas import tpu_sc as plsc

N = 10000
E = 320000
H = 128
R = 4
NB = 4
L = 3
G = 16
EPS = 1e-5

NC = 2
NS = 16
NW = NC * NS

NPAD = 10240
NRP = 40960
CHUNK = 128
EPW = E // NW
ROWS_PER_TILE = NPAD // NS
NCH = 80
BLK = 16
EP = NW * NCH * CHUNK
CCH = 157
CP = NS * CCH * CHUNK

_sc_mesh = plsc.VectorSubcoreMesh(core_axis_name="c", subcore_axis_name="s")
_sc_params = pltpu.CompilerParams(needs_layout_passes=False)


@functools.partial(
    pl.kernel,
    out_type=(
        jax.ShapeDtypeStruct((NPAD, H), jnp.float32),
        jax.ShapeDtypeStruct((E,), jnp.float32),
    ),
    mesh=_sc_mesh,
    scratch_types=[
        pltpu.VMEM_SHARED((NRP,), jnp.float32),
        pltpu.VMEM((CHUNK,), jnp.int32),
        pltpu.VMEM((CHUNK, H), jnp.float32),
        pltpu.VMEM((CHUNK,), jnp.float32),
        pltpu.VMEM((NRP // NS,), jnp.float32),
        pltpu.VMEM((NRP,), jnp.float32),
        pltpu.VMEM((CCH, CHUNK), jnp.int32),
        pltpu.VMEM((EPW,), jnp.int32),
        pltpu.VMEM((EPW,), jnp.float32),
        pltpu.SemaphoreType.DMA,
    ],
    compiler_params=_sc_params,
)
def _prep(xids_hbm, cidx3_hbm, cidx_hbm, emb_hbm, x_hbm, w_hbm,
          cnt_sh, idbuf, rowbuf, onesbuf, slbuf, recipbuf, cq2, cqbuf,
          woutbuf, sem):
    cid = lax.axis_index("c")
    sid = lax.axis_index("s")
    wid = sid * NC + cid

    slice_len = NRP // NS

    def zstore(i, _):
        slbuf[pl.ds(i * 16, 16)] = jnp.zeros((16,), jnp.float32)
        return 0
    lax.fori_loop(0, slice_len // 16, zstore, 0)
    pltpu.sync_copy(slbuf, cnt_sh.at[pl.ds(sid * slice_len, slice_len)])

    def ostore(i, _):
        onesbuf[pl.ds(i * 16, 16)] = jnp.ones((16,), jnp.float32)
        return 0
    lax.fori_loop(0, CHUNK // 16, ostore, 0)

    nemb = NPAD // CHUNK
    def embody(j, _):
        c = wid + NW * j
        @pl.when(c < nemb)
        def _():
            base = c * CHUNK
            pltpu.sync_copy(xids_hbm.at[pl.ds(base, CHUNK)], idbuf)
            pltpu.async_copy(emb_hbm.at[idbuf], rowbuf, sem).wait()
            pltpu.sync_copy(rowbuf, x_hbm.at[pl.ds(base, CHUNK)])
        return 0
    lax.fori_loop(0, (nemb + NW - 1) // NW, embody, 0)

    plsc.subcore_barrier()

    pltpu.sync_copy(cidx3_hbm.at[sid], cq2)
    def cbody(j, _):
        pltpu.sync_copy(onesbuf, cnt_sh.at[cq2.at[j]], add=True)
        return 0
    lax.fori_loop(0, CCH, cbody, 0)

    plsc.subcore_barrier()

    sl = pl.ds(sid * slice_len, slice_len)
    pltpu.sync_copy(cnt_sh.at[sl], slbuf)
    def rbody(i, _):
        v = slbuf[pl.ds(i * 16, 16)]
        slbuf[pl.ds(i * 16, 16)] = 1.0 / jnp.maximum(v, 1.0)
        return 0
    lax.fori_loop(0, slice_len // 16, rbody, 0)
    pltpu.sync_copy(slbuf, cnt_sh.at[sl])

    plsc.subcore_barrier()

    pltpu.sync_copy(cnt_sh, recipbuf)
    wbase = wid * EPW
    pltpu.sync_copy(cidx_hbm.at[pl.ds(wbase, EPW)], cqbuf)
    def wbody(i, _):
        idxv = cqbuf[pl.ds(i * 16, 16)]
        woutbuf[pl.ds(i * 16, 16)] = plsc.load_gather(recipbuf, [idxv])
        return 0
    lax.fori_loop(0, EPW // 16, wbody, 0)
    pltpu.sync_copy(woutbuf, w_hbm.at[pl.ds(wbase, EPW)])


@functools.partial(
    pl.kernel,
    out_type=jax.ShapeDtypeStruct((NC, NPAD, H), jnp.float32),
    mesh=_sc_mesh,
    scratch_types=[
        pltpu.VMEM_SHARED((NPAD, H), jnp.float32),
        pltpu.VMEM((BLK, CHUNK), jnp.int32),
        pltpu.VMEM((BLK, CHUNK), jnp.int32),
        pltpu.VMEM((BLK, CHUNK), jnp.float32),
        pltpu.VMEM((CHUNK, H), jnp.float32),
        pltpu.VMEM((CHUNK, H), jnp.float32),
        pltpu.SemaphoreType.DMA,
        pltpu.SemaphoreType.DMA,
    ],
    compiler_params=_sc_params,
)
def _edge_agg(xall_hbm, gidx_hbm, dst_hbm, w_hbm, out_hbm,
              acc_sh, gi2, di2, w2, rows0, rows1, sem0, sem1):
    cid = lax.axis_index("c")
    sid = lax.axis_index("s")
    wid = sid * NC + cid

    def zrow(i, _):
        def zcol(i16, _):
            rows0[i, pl.ds(i16 * 16, 16)] = jnp.zeros((16,), jnp.float32)
            return 0
        lax.fori_loop(0, H // 16, zcol, 0)
        return 0
    lax.fori_loop(0, CHUNK, zrow, 0)
    row0 = sid * ROWS_PER_TILE
    for p in range(ROWS_PER_TILE // CHUNK):
        pltpu.sync_copy(rows0, acc_sh.at[pl.ds(row0 + p * CHUNK, CHUNK)])

    plsc.subcore_barrier()

    def blkbody(bi, _):
        bsl = pl.ds(bi * BLK, BLK)
        pltpu.sync_copy(gidx_hbm.at[wid, bsl], gi2)
        pltpu.sync_copy(dst_hbm.at[wid, bsl], di2)
        pltpu.sync_copy(w_hbm.at[wid, bsl], w2)
        pltpu.async_copy(xall_hbm.at[gi2.at[0]], rows0, sem0)
        pltpu.async_copy(xall_hbm.at[gi2.at[1]], rows1, sem1)
        def ebody(jj, _):
            for b, rows, sem in ((0, rows0, sem0), (1, rows1, sem1)):
                j = 2 * jj + b
                pltpu.make_async_copy(xall_hbm.at[gi2.at[j]], rows,
                                      sem).wait()
                def sgroup(k16):
                    wg = w2[j, pl.ds(k16 * 16, 16)]
                    for i in range(16):
                        k = k16 * 16 + i
                        sb = jnp.full((16,), wg[i], jnp.float32)
                        for i16 in range(H // 16):
                            rows[k, pl.ds(i16 * 16, 16)] = (
                                rows[k, pl.ds(i16 * 16, 16)] * sb)
                plsc.parallel_loop(0, CHUNK // 16, 1, unroll=2)(sgroup)
                pltpu.sync_copy(rows, acc_sh.at[di2.at[j]], add=True)
                @pl.when(jj < BLK // 2 - 1)
                def _():
                    pltpu.async_copy(xall_hbm.at[gi2.at[j + 2]], rows, sem)
            return 0
        lax.fori_loop(0, BLK // 2, ebody, 0)
        return 0
    lax.fori_loop(0, NCH // BLK, blkbody, 0)

    plsc.subcore_barrier()
    pltpu.sync_copy(acc_sh.at[pl.ds(row0, ROWS_PER_TILE)],
                    out_hbm.at[cid, pl.ds(row0, ROWS_PER_TILE)])


def _mm_body(comp_ref, bases_ref, root_ref, x_ref, xall_ref, xroot_ref):
    x = x_ref[...]
    for r in range(R):
        w = comp_ref[r, 0] * bases_ref[0]
        for b in range(1, NB):
            w = w + comp_ref[r, b] * bases_ref[b]
        xall_ref[:, r * H:(r + 1) * H] = jnp.dot(
            x, w, preferred_element_type=jnp.float32)
    xroot_ref[...] = jnp.dot(x, root_ref[...],
                             preferred_element_type=jnp.float32)


_BM = 1024
_mm = pl.pallas_call(
    _mm_body,
    grid=(NPAD // _BM,),
    in_specs=[
        pl.BlockSpec(memory_space=pltpu.SMEM),
        pl.BlockSpec((NB, H, H), lambda i: (0, 0, 0)),
        pl.BlockSpec((H, H), lambda i: (0, 0)),
        pl.BlockSpec((_BM, H), lambda i: (i, 0)),
    ],
    out_specs=[
        pl.BlockSpec((_BM, R * H), lambda i: (i, 0)),
        pl.BlockSpec((_BM, H), lambda i: (i, 0)),
    ],
    out_shape=[
        jax.ShapeDtypeStruct((NPAD, R * H), jnp.float32),
        jax.ShapeDtypeStruct((NPAD, H), jnp.float32),
    ],
)


def _combine_body(part_ref, xroot_ref, x_ref, bias_ref, bnw_ref, bnb_ref,
                  out_ref):
    h = part_ref[0] + part_ref[1] + xroot_ref[...] + bias_ref[...]
    rows = lax.broadcasted_iota(jnp.int32, (NPAD, 1), 0)
    m = rows < N
    hm = jnp.where(m, h, 0.0)
    mu = jnp.sum(hm, axis=0, keepdims=True) * (1.0 / N)
    d = h - mu
    var = jnp.sum(jnp.where(m, d * d, 0.0), axis=0, keepdims=True) * (1.0 / N)
    hn = d / jnp.sqrt(var + EPS) * bnw_ref[...] + bnb_ref[...]
    out_ref[...] = x_ref[...] + jnp.maximum(hn, 0.0)


_combine = pl.pallas_call(
    _combine_body,
    out_shape=jax.ShapeDtypeStruct((NPAD, H), jnp.float32),
)


def _pool_body(x_ref, batch_ref, l1m_ref, l1x_ref, l1b_ref, lo_ref, lob_ref,
               t_ref, out_ref, xm_ref, xx_ref):
    x = x_ref[...]
    b = batch_ref[...]
    for g in range(G):
        m = b == g
        cnt = jnp.sum(jnp.where(m, 1.0, 0.0))
        s = jnp.sum(jnp.where(m, x, 0.0), axis=0)
        xm_ref[g, :] = s / jnp.maximum(cnt, 1.0)
        mx = jnp.max(jnp.where(m, x, -jnp.inf), axis=0)
        xx_ref[g, :] = jnp.where(mx > -1e37, mx, 0.0)
    hidden = jnp.maximum(
        jnp.dot(xm_ref[...], l1m_ref[...], preferred_element_type=jnp.float32)
        + jnp.dot(xx_ref[...], l1x_ref[...], preferred_element_type=jnp.float32)
        + l1b_ref[...], 0.0)
    logits = jnp.dot(hidden, lo_ref[...],
                     preferred_element_type=jnp.float32) + lob_ref[...]
    t = jnp.maximum(t_ref[0, 0], 1e-4)
    out_ref[...] = logits / t


_pool = pl.pallas_call(
    _pool_body,
    in_specs=[
        pl.BlockSpec(memory_space=pltpu.VMEM),
        pl.BlockSpec(memory_space=pltpu.VMEM),
        pl.BlockSpec(memory_space=pltpu.VMEM),
        pl.BlockSpec(memory_space=pltpu.VMEM),
        pl.BlockSpec(memory_space=pltpu.VMEM),
        pl.BlockSpec(memory_space=pltpu.VMEM),
        pl.BlockSpec(memory_space=pltpu.VMEM),
        pl.BlockSpec(memory_space=pltpu.SMEM),
    ],
    out_shape=jax.ShapeDtypeStruct((G, H), jnp.float32),
    scratch_shapes=[
        pltpu.VMEM((G, H), jnp.float32),
        pltpu.VMEM((G, H), jnp.float32),
    ],
)


def kernel(x_ids, edge_index, edge_type, batch, emb, bases, comp, root,
           conv_bias, bn_w, bn_b, lin1_w, lin1_b, lout_w, lout_b,
           temperature):
    src = edge_index[0].astype(jnp.int32)
    dst = edge_index[1].astype(jnp.int32)
    et = edge_type.astype(jnp.int32)
    gidx = src * R + et
    cidx = dst * R + et
    xids_pad = jnp.pad(x_ids.astype(jnp.int32), (0, NPAD - N))
    cidx3 = jnp.pad(cidx, (0, CP - E),
                    constant_values=NRP - 1).reshape(NS, CCH, CHUNK)

    x, w = _prep(xids_pad, cidx3, cidx, emb)

    pad_idx = jnp.arange(EP - E, dtype=jnp.int32)
    gidx3 = jnp.concatenate([gidx, pad_idx % (N * R)]).reshape(NW, NCH, CHUNK)
    dst3 = jnp.concatenate([dst, pad_idx % N]).reshape(NW, NCH, CHUNK)
    w3 = jnp.concatenate(
        [w, jnp.zeros((EP - E,), jnp.float32)]).reshape(NW, NCH, CHUNK)

    for l in range(L):
        xall, xroot = _mm(comp[l], bases[l], root[l], x)
        part = _edge_agg(xall.reshape(NPAD * R, H), gidx3, dst3, w3)
        x = _combine(part, xroot, x, conv_bias[l].reshape(1, H),
                     bn_w[l].reshape(1, H), bn_b[l].reshape(1, H))

    batch_pad = jnp.pad(batch.astype(jnp.int32), (0, NPAD - N),
                        constant_values=G).reshape(NPAD, 1)
    l1t = lin1_w.T
    lo_pad = jnp.pad(lout_w.T, ((0, 0), (0, H - 2)))
    lob_pad = jnp.pad(lout_b, (0, H - 2)).reshape(1, H)
    out = _pool(x, batch_pad, l1t[:H], l1t[H:], lin1_b.reshape(1, H),
                lo_pad, lob_pad, temperature.reshape(1, 1))
    return out[:, :2]


# speedup vs baseline: 1.0391x; 1.0391x over previous
import functools

import jax
import jax.numpy as jnp
from jax import lax
from jax.experimental import pallas as pl
from jax.experimental.pallas import tpu as pltpu
from jax.experimental.pallas import tpu_sc as plsc

N = 10000
E = 320000
H = 128
R = 4
NB = 4
L = 3
G = 16
EPS = 1e-5

NC = 2
NS = 16
NW = NC * NS

NPAD = 10240
NRP = 40960
CHUNK = 128
EPW = E // NW
ROWS_PER_TILE = NPAD // NS
NCH = 80
BLK = 16
EP = NW * NCH * CHUNK
CCH = 157
CP = NS * CCH * CHUNK

_sc_mesh = plsc.VectorSubcoreMesh(core_axis_name="c", subcore_axis_name="s")
_sc_params = pltpu.CompilerParams(needs_layout_passes=False)


@functools.partial(
    pl.kernel,
    out_type=(
        jax.ShapeDtypeStruct((NPAD, H), jnp.float32),
        jax.ShapeDtypeStruct((E,), jnp.float32),
    ),
    mesh=_sc_mesh,
    scratch_types=[
        pltpu.VMEM_SHARED((NRP,), jnp.float32),
        pltpu.VMEM((CHUNK,), jnp.int32),
        pltpu.VMEM((CHUNK, H), jnp.float32),
        pltpu.VMEM((CHUNK,), jnp.float32),
        pltpu.VMEM((NRP // NS,), jnp.float32),
        pltpu.VMEM((NRP,), jnp.float32),
        pltpu.VMEM((CCH, CHUNK), jnp.int32),
        pltpu.VMEM((EPW,), jnp.int32),
        pltpu.VMEM((EPW,), jnp.float32),
        pltpu.SemaphoreType.DMA,
    ],
    compiler_params=_sc_params,
)
def _prep(xids_hbm, cidx3_hbm, cidx_hbm, emb_hbm, x_hbm, w_hbm,
          cnt_sh, idbuf, rowbuf, onesbuf, slbuf, recipbuf, cq2, cqbuf,
          woutbuf, sem):
    cid = lax.axis_index("c")
    sid = lax.axis_index("s")
    wid = sid * NC + cid

    slice_len = NRP // NS

    def zstore(i, _):
        slbuf[pl.ds(i * 16, 16)] = jnp.zeros((16,), jnp.float32)
        return 0
    lax.fori_loop(0, slice_len // 16, zstore, 0)
    pltpu.sync_copy(slbuf, cnt_sh.at[pl.ds(sid * slice_len, slice_len)])

    def ostore(i, _):
        onesbuf[pl.ds(i * 16, 16)] = jnp.ones((16,), jnp.float32)
        return 0
    lax.fori_loop(0, CHUNK // 16, ostore, 0)

    nemb = NPAD // CHUNK
    def embody(j, _):
        c = wid + NW * j
        @pl.when(c < nemb)
        def _():
            base = c * CHUNK
            pltpu.sync_copy(xids_hbm.at[pl.ds(base, CHUNK)], idbuf)
            pltpu.async_copy(emb_hbm.at[idbuf], rowbuf, sem).wait()
            pltpu.sync_copy(rowbuf, x_hbm.at[pl.ds(base, CHUNK)])
        return 0
    lax.fori_loop(0, (nemb + NW - 1) // NW, embody, 0)

    plsc.subcore_barrier()

    pltpu.sync_copy(cidx3_hbm.at[sid], cq2)
    def cbody(j, _):
        pltpu.sync_copy(onesbuf, cnt_sh.at[cq2.at[j]], add=True)
        return 0
    lax.fori_loop(0, CCH, cbody, 0)

    plsc.subcore_barrier()

    sl = pl.ds(sid * slice_len, slice_len)
    pltpu.sync_copy(cnt_sh.at[sl], slbuf)
    def rbody(i, _):
        v = slbuf[pl.ds(i * 16, 16)]
        slbuf[pl.ds(i * 16, 16)] = 1.0 / jnp.maximum(v, 1.0)
        return 0
    lax.fori_loop(0, slice_len // 16, rbody, 0)
    pltpu.sync_copy(slbuf, cnt_sh.at[sl])

    plsc.subcore_barrier()

    pltpu.sync_copy(cnt_sh, recipbuf)
    wbase = wid * EPW
    pltpu.sync_copy(cidx_hbm.at[pl.ds(wbase, EPW)], cqbuf)
    def wbody(i, _):
        idxv = cqbuf[pl.ds(i * 16, 16)]
        woutbuf[pl.ds(i * 16, 16)] = plsc.load_gather(recipbuf, [idxv])
        return 0
    lax.fori_loop(0, EPW // 16, wbody, 0)
    pltpu.sync_copy(woutbuf, w_hbm.at[pl.ds(wbase, EPW)])


@functools.partial(
    pl.kernel,
    out_type=jax.ShapeDtypeStruct((NC, NPAD, H), jnp.float32),
    mesh=_sc_mesh,
    scratch_types=[
        pltpu.VMEM_SHARED((NPAD, H), jnp.float32),
        pltpu.VMEM((BLK, CHUNK), jnp.int32),
        pltpu.VMEM((BLK, CHUNK), jnp.int32),
        pltpu.VMEM((BLK, CHUNK), jnp.float32),
        pltpu.VMEM((CHUNK, H), jnp.float32),
        pltpu.VMEM((CHUNK, H), jnp.float32),
        pltpu.SemaphoreType.DMA,
        pltpu.SemaphoreType.DMA,
    ],
    compiler_params=_sc_params,
)
def _edge_agg(xall_hbm, gidx_hbm, dst_hbm, w_hbm, out_hbm,
              acc_sh, gi2, di2, w2, rows0, rows1, sem0, sem1):
    cid = lax.axis_index("c")
    sid = lax.axis_index("s")
    wid = sid * NC + cid

    def zrow(i, _):
        def zcol(i16, _):
            rows0[i, pl.ds(i16 * 16, 16)] = jnp.zeros((16,), jnp.float32)
            return 0
        lax.fori_loop(0, H // 16, zcol, 0)
        return 0
    lax.fori_loop(0, CHUNK, zrow, 0)
    row0 = sid * ROWS_PER_TILE
    for p in range(ROWS_PER_TILE // CHUNK):
        pltpu.sync_copy(rows0, acc_sh.at[pl.ds(row0 + p * CHUNK, CHUNK)])

    plsc.subcore_barrier()

    def blkbody(bi, _):
        bsl = pl.ds(bi * BLK, BLK)
        pltpu.sync_copy(gidx_hbm.at[wid, bsl], gi2)
        pltpu.sync_copy(dst_hbm.at[wid, bsl], di2)
        pltpu.sync_copy(w_hbm.at[wid, bsl], w2)
        pltpu.async_copy(xall_hbm.at[gi2.at[0]], rows0, sem0)
        pltpu.async_copy(xall_hbm.at[gi2.at[1]], rows1, sem1)
        def ebody(jj, _):
            for b, rows, sem in ((0, rows0, sem0), (1, rows1, sem1)):
                j = 2 * jj + b
                pltpu.make_async_copy(xall_hbm.at[gi2.at[j]], rows,
                                      sem).wait()
                def sgroup(k16):
                    wg = w2[j, pl.ds(k16 * 16, 16)]
                    for i in range(16):
                        k = k16 * 16 + i
                        sb = jnp.full((16,), wg[i], jnp.float32)
                        for i16 in range(H // 16):
                            rows[k, pl.ds(i16 * 16, 16)] = (
                                rows[k, pl.ds(i16 * 16, 16)] * sb)
                plsc.parallel_loop(0, CHUNK // 16, 1, unroll=2)(sgroup)
                pltpu.sync_copy(rows, acc_sh.at[di2.at[j]], add=True)
                @pl.when(jj < BLK // 2 - 1)
                def _():
                    pltpu.async_copy(xall_hbm.at[gi2.at[j + 2]], rows, sem)
            return 0
        lax.fori_loop(0, BLK // 2, ebody, 0)
        return 0
    lax.fori_loop(0, NCH // BLK, blkbody, 0)

    plsc.subcore_barrier()
    pltpu.sync_copy(acc_sh.at[pl.ds(row0, ROWS_PER_TILE)],
                    out_hbm.at[cid, pl.ds(row0, ROWS_PER_TILE)])


def _mm_body(comp_ref, bases_ref, root_ref, x_ref, xall_ref, xroot_ref):
    x = x_ref[...]
    for r in range(R):
        w = comp_ref[r, 0] * bases_ref[0]
        for b in range(1, NB):
            w = w + comp_ref[r, b] * bases_ref[b]
        xall_ref[:, r * H:(r + 1) * H] = jnp.dot(
            x, w, preferred_element_type=jnp.float32)
    xroot_ref[...] = jnp.dot(x, root_ref[...],
                             preferred_element_type=jnp.float32)


_BM = 1024
_mm = pl.pallas_call(
    _mm_body,
    grid=(NPAD // _BM,),
    in_specs=[
        pl.BlockSpec(memory_space=pltpu.SMEM),
        pl.BlockSpec((NB, H, H), lambda i: (0, 0, 0)),
        pl.BlockSpec((H, H), lambda i: (0, 0)),
        pl.BlockSpec((_BM, H), lambda i: (i, 0)),
    ],
    out_specs=[
        pl.BlockSpec((_BM, R * H), lambda i: (i, 0)),
        pl.BlockSpec((_BM, H), lambda i: (i, 0)),
    ],
    out_shape=[
        jax.ShapeDtypeStruct((NPAD, R * H), jnp.float32),
        jax.ShapeDtypeStruct((NPAD, H), jnp.float32),
    ],
)


def _bn_relu_residual(part_ref, xroot_ref, x_ref, bias_ref, bnw_ref, bnb_ref):
    h = part_ref[0] + part_ref[1] + xroot_ref[...] + bias_ref[...]
    rows = lax.broadcasted_iota(jnp.int32, (NPAD, 1), 0)
    m = rows < N
    hm = jnp.where(m, h, 0.0)
    mu = jnp.sum(hm, axis=0, keepdims=True) * (1.0 / N)
    d = h - mu
    var = jnp.sum(jnp.where(m, d * d, 0.0), axis=0, keepdims=True) * (1.0 / N)
    hn = d / jnp.sqrt(var + EPS) * bnw_ref[...] + bnb_ref[...]
    return x_ref[...] + jnp.maximum(hn, 0.0)


def _mmc_body(part_ref, xroot_ref, x_ref, bias_ref, bnw_ref, bnb_ref,
              comp_ref, bases_ref, root_ref, xnew_ref, xall_ref, xroot2_ref):
    xn = _bn_relu_residual(part_ref, xroot_ref, x_ref, bias_ref, bnw_ref,
                           bnb_ref)
    xnew_ref[...] = xn
    for r in range(R):
        w = comp_ref[r, 0] * bases_ref[0]
        for b in range(1, NB):
            w = w + comp_ref[r, b] * bases_ref[b]
        xall_ref[:, r * H:(r + 1) * H] = jnp.dot(
            xn, w, preferred_element_type=jnp.float32)
    xroot2_ref[...] = jnp.dot(xn, root_ref[...],
                              preferred_element_type=jnp.float32)


_mmc = pl.pallas_call(
    _mmc_body,
    in_specs=[
        pl.BlockSpec(memory_space=pltpu.VMEM),
        pl.BlockSpec(memory_space=pltpu.VMEM),
        pl.BlockSpec(memory_space=pltpu.VMEM),
        pl.BlockSpec(memory_space=pltpu.VMEM),
        pl.BlockSpec(memory_space=pltpu.VMEM),
        pl.BlockSpec(memory_space=pltpu.VMEM),
        pl.BlockSpec(memory_space=pltpu.SMEM),
        pl.BlockSpec(memory_space=pltpu.VMEM),
        pl.BlockSpec(memory_space=pltpu.VMEM),
    ],
    out_shape=[
        jax.ShapeDtypeStruct((NPAD, H), jnp.float32),
        jax.ShapeDtypeStruct((NPAD, R * H), jnp.float32),
        jax.ShapeDtypeStruct((NPAD, H), jnp.float32),
    ],
)


def _pool_body(part_ref, xroot_ref, x_ref, bias_ref, bnw_ref, bnb_ref,
               batch_ref, l1m_ref, l1x_ref, l1b_ref, lo_ref, lob_ref,
               t_ref, out_ref, xm_ref, xx_ref):
    x = _bn_relu_residual(part_ref, xroot_ref, x_ref, bias_ref, bnw_ref,
                          bnb_ref)
    b = batch_ref[...]
    for g in range(G):
        m = b == g
        cnt = jnp.sum(jnp.where(m, 1.0, 0.0))
        s = jnp.sum(jnp.where(m, x, 0.0), axis=0)
        xm_ref[g, :] = s / jnp.maximum(cnt, 1.0)
        mx = jnp.max(jnp.where(m, x, -jnp.inf), axis=0)
        xx_ref[g, :] = jnp.where(mx > -1e37, mx, 0.0)
    hidden = jnp.maximum(
        jnp.dot(xm_ref[...], l1m_ref[...], preferred_element_type=jnp.float32)
        + jnp.dot(xx_ref[...], l1x_ref[...], preferred_element_type=jnp.float32)
        + l1b_ref[...], 0.0)
    logits = jnp.dot(hidden, lo_ref[...],
                     preferred_element_type=jnp.float32) + lob_ref[...]
    t = jnp.maximum(t_ref[0, 0], 1e-4)
    out_ref[...] = logits / t


_pool = pl.pallas_call(
    _pool_body,
    in_specs=[
        pl.BlockSpec(memory_space=pltpu.VMEM),
        pl.BlockSpec(memory_space=pltpu.VMEM),
        pl.BlockSpec(memory_space=pltpu.VMEM),
        pl.BlockSpec(memory_space=pltpu.VMEM),
        pl.BlockSpec(memory_space=pltpu.VMEM),
        pl.BlockSpec(memory_space=pltpu.VMEM),
        pl.BlockSpec(memory_space=pltpu.VMEM),
        pl.BlockSpec(memory_space=pltpu.VMEM),
        pl.BlockSpec(memory_space=pltpu.VMEM),
        pl.BlockSpec(memory_space=pltpu.VMEM),
        pl.BlockSpec(memory_space=pltpu.VMEM),
        pl.BlockSpec(memory_space=pltpu.VMEM),
        pl.BlockSpec(memory_space=pltpu.SMEM),
    ],
    out_shape=jax.ShapeDtypeStruct((G, H), jnp.float32),
    scratch_shapes=[
        pltpu.VMEM((G, H), jnp.float32),
        pltpu.VMEM((G, H), jnp.float32),
    ],
)


def kernel(x_ids, edge_index, edge_type, batch, emb, bases, comp, root,
           conv_bias, bn_w, bn_b, lin1_w, lin1_b, lout_w, lout_b,
           temperature):
    src = edge_index[0].astype(jnp.int32)
    dst = edge_index[1].astype(jnp.int32)
    et = edge_type.astype(jnp.int32)
    gidx = src * R + et
    cidx = dst * R + et
    xids_pad = jnp.pad(x_ids.astype(jnp.int32), (0, NPAD - N))
    cidx3 = jnp.pad(cidx, (0, CP - E),
                    constant_values=NRP - 1).reshape(NS, CCH, CHUNK)

    x, w = _prep(xids_pad, cidx3, cidx, emb)

    pad_idx = jnp.arange(EP - E, dtype=jnp.int32)
    gidx3 = jnp.concatenate([gidx, pad_idx % (N * R)]).reshape(NW, NCH, CHUNK)
    dst3 = jnp.concatenate([dst, pad_idx % N]).reshape(NW, NCH, CHUNK)
    w3 = jnp.concatenate(
        [w, jnp.zeros((EP - E,), jnp.float32)]).reshape(NW, NCH, CHUNK)

    xall, xroot = _mm(comp[0], bases[0], root[0], x)
    for l in range(L - 1):
        part = _edge_agg(xall.reshape(NPAD * R, H), gidx3, dst3, w3)
        x, xall, xroot = _mmc(part, xroot, x, conv_bias[l].reshape(1, H),
                              bn_w[l].reshape(1, H), bn_b[l].reshape(1, H),
                              comp[l + 1], bases[l + 1], root[l + 1])
    part = _edge_agg(xall.reshape(NPAD * R, H), gidx3, dst3, w3)

    batch_pad = jnp.pad(batch.astype(jnp.int32), (0, NPAD - N),
                        constant_values=G).reshape(NPAD, 1)
    l1t = lin1_w.T
    lo_pad = jnp.pad(lout_w.T, ((0, 0), (0, H - 2)))
    lob_pad = jnp.pad(lout_b, (0, H - 2)).reshape(1, H)
    out = _pool(part, xroot, x, conv_bias[L - 1].reshape(1, H),
                bn_w[L - 1].reshape(1, H), bn_b[L - 1].reshape(1, H),
                batch_pad, l1t[:H], l1t[H:], lin1_b.reshape(1, H),
                lo_pad, lob_pad, temperature.reshape(1, 1))
    return out[:, :2]


# speedup vs baseline: 1.0455x; 1.0062x over previous
import functools

import jax
import jax.numpy as jnp
from jax import lax
from jax.experimental import pallas as pl
from jax.experimental.pallas import tpu as pltpu
from jax.experimental.pallas import tpu_sc as plsc

N = 10000
E = 320000
H = 128
R = 4
NB = 4
L = 3
G = 16
EPS = 1e-5

NC = 2
NS = 16
NW = NC * NS

NPAD = 10240
NRP = 40960
CHUNK = 128
EPW = E // NW
ROWS_PER_TILE = NPAD // NS
NCH = 80
BLK = 16
EP = NW * NCH * CHUNK
CCH = 157
CP = NS * CCH * CHUNK

_sc_mesh = plsc.VectorSubcoreMesh(core_axis_name="c", subcore_axis_name="s")
_sc_params = pltpu.CompilerParams(needs_layout_passes=False)


@functools.partial(
    pl.kernel,
    out_type=(
        jax.ShapeDtypeStruct((NPAD, H), jnp.float32),
        jax.ShapeDtypeStruct((E,), jnp.float32),
    ),
    mesh=_sc_mesh,
    scratch_types=[
        pltpu.VMEM_SHARED((NRP,), jnp.float32),
        pltpu.VMEM((CHUNK,), jnp.int32),
        pltpu.VMEM((CHUNK, H), jnp.float32),
        pltpu.VMEM((CHUNK,), jnp.float32),
        pltpu.VMEM((NRP // NS,), jnp.float32),
        pltpu.VMEM((NRP,), jnp.float32),
        pltpu.VMEM((CCH, CHUNK), jnp.int32),
        pltpu.VMEM((EPW,), jnp.int32),
        pltpu.VMEM((EPW,), jnp.float32),
        pltpu.SemaphoreType.DMA,
    ],
    compiler_params=_sc_params,
)
def _prep(xids_hbm, cidx3_hbm, cidx_hbm, emb_hbm, x_hbm, w_hbm,
          cnt_sh, idbuf, rowbuf, onesbuf, slbuf, recipbuf, cq2, cqbuf,
          woutbuf, sem):
    cid = lax.axis_index("c")
    sid = lax.axis_index("s")
    wid = sid * NC + cid

    slice_len = NRP // NS

    def zstore(i, _):
        slbuf[pl.ds(i * 16, 16)] = jnp.zeros((16,), jnp.float32)
        return 0
    lax.fori_loop(0, slice_len // 16, zstore, 0)
    pltpu.sync_copy(slbuf, cnt_sh.at[pl.ds(sid * slice_len, slice_len)])

    def ostore(i, _):
        onesbuf[pl.ds(i * 16, 16)] = jnp.ones((16,), jnp.float32)
        return 0
    lax.fori_loop(0, CHUNK // 16, ostore, 0)

    nemb = NPAD // CHUNK
    def embody(j, _):
        c = wid + NW * j
        @pl.when(c < nemb)
        def _():
            base = c * CHUNK
            pltpu.sync_copy(xids_hbm.at[pl.ds(base, CHUNK)], idbuf)
            pltpu.async_copy(emb_hbm.at[idbuf], rowbuf, sem).wait()
            pltpu.sync_copy(rowbuf, x_hbm.at[pl.ds(base, CHUNK)])
        return 0
    lax.fori_loop(0, (nemb + NW - 1) // NW, embody, 0)

    plsc.subcore_barrier()

    pltpu.sync_copy(cidx3_hbm.at[sid], cq2)
    def cbody(j, _):
        pltpu.sync_copy(onesbuf, cnt_sh.at[cq2.at[j]], add=True)
        return 0
    lax.fori_loop(0, CCH, cbody, 0)

    plsc.subcore_barrier()

    sl = pl.ds(sid * slice_len, slice_len)
    pltpu.sync_copy(cnt_sh.at[sl], slbuf)
    def rbody(i, _):
        v = slbuf[pl.ds(i * 16, 16)]
        slbuf[pl.ds(i * 16, 16)] = 1.0 / jnp.maximum(v, 1.0)
        return 0
    lax.fori_loop(0, slice_len // 16, rbody, 0)
    pltpu.sync_copy(slbuf, cnt_sh.at[sl])

    plsc.subcore_barrier()

    pltpu.sync_copy(cnt_sh, recipbuf)
    wbase = wid * EPW
    pltpu.sync_copy(cidx_hbm.at[pl.ds(wbase, EPW)], cqbuf)
    def wbody(i, _):
        idxv = cqbuf[pl.ds(i * 16, 16)]
        woutbuf[pl.ds(i * 16, 16)] = plsc.load_gather(recipbuf, [idxv])
        return 0
    lax.fori_loop(0, EPW // 16, wbody, 0)
    pltpu.sync_copy(woutbuf, w_hbm.at[pl.ds(wbase, EPW)])


@functools.partial(
    pl.kernel,
    out_type=jax.ShapeDtypeStruct((NC, NPAD, H), jnp.float32),
    mesh=_sc_mesh,
    scratch_types=[
        pltpu.VMEM_SHARED((NPAD, H), jnp.float32),
        pltpu.VMEM((BLK, CHUNK), jnp.int32),
        pltpu.VMEM((2 * BLK, CHUNK // 2), jnp.int32),
        pltpu.VMEM((BLK, CHUNK), jnp.float32),
        pltpu.VMEM((CHUNK, H), jnp.float32),
        pltpu.VMEM((CHUNK, H), jnp.float32),
        pltpu.SemaphoreType.DMA,
        pltpu.SemaphoreType.DMA,
        pltpu.SemaphoreType.DMA,
    ],
    compiler_params=_sc_params,
)
def _edge_agg(xall_hbm, gidx_hbm, dst_hbm, w_hbm, out_hbm,
              acc_sh, gi2, di2, w2, rows0, rows1, sem0, sem1, sem_s):
    cid = lax.axis_index("c")
    sid = lax.axis_index("s")
    wid = sid * NC + cid

    def zrow(i, _):
        def zcol(i16, _):
            rows0[i, pl.ds(i16 * 16, 16)] = jnp.zeros((16,), jnp.float32)
            return 0
        lax.fori_loop(0, H // 16, zcol, 0)
        return 0
    lax.fori_loop(0, CHUNK, zrow, 0)
    row0 = sid * ROWS_PER_TILE
    for p in range(ROWS_PER_TILE // CHUNK):
        pltpu.sync_copy(rows0, acc_sh.at[pl.ds(row0 + p * CHUNK, CHUNK)])

    plsc.subcore_barrier()

    def blkbody(bi, _):
        pltpu.sync_copy(gidx_hbm.at[wid, pl.ds(bi * BLK, BLK)], gi2)
        pltpu.sync_copy(dst_hbm.at[wid, pl.ds(bi * 2 * BLK, 2 * BLK)], di2)
        pltpu.sync_copy(w_hbm.at[wid, pl.ds(bi * BLK, BLK)], w2)
        pltpu.async_copy(xall_hbm.at[gi2.at[0]], rows0, sem0)
        pltpu.async_copy(xall_hbm.at[gi2.at[1]], rows1, sem1)
        def ebody(jj, _):
            for b, rows, sem in ((0, rows0, sem0), (1, rows1, sem1)):
                j = 2 * jj + b
                pltpu.make_async_copy(xall_hbm.at[gi2.at[j]], rows,
                                      sem).wait()
                def scale_half(lo):
                    def sgroup(k16):
                        wg = w2[j, pl.ds(k16 * 16, 16)]
                        for i in range(16):
                            k = k16 * 16 + i
                            sb = jnp.full((16,), wg[i], jnp.float32)
                            for i16 in range(H // 16):
                                rows[k, pl.ds(i16 * 16, 16)] = (
                                    rows[k, pl.ds(i16 * 16, 16)] * sb)
                    plsc.parallel_loop(lo, lo + CHUNK // 32, 1,
                                       unroll=2)(sgroup)
                scale_half(0)
                desc = pltpu.async_copy(rows.at[pl.ds(0, CHUNK // 2)],
                                        acc_sh.at[di2.at[2 * j]], sem_s,
                                        add=True)
                scale_half(CHUNK // 32)
                pltpu.sync_copy(rows.at[pl.ds(CHUNK // 2, CHUNK // 2)],
                                acc_sh.at[di2.at[2 * j + 1]], add=True)
                desc.wait()
                @pl.when(jj < BLK // 2 - 1)
                def _():
                    pltpu.async_copy(xall_hbm.at[gi2.at[j + 2]], rows, sem)
            return 0
        lax.fori_loop(0, BLK // 2, ebody, 0)
        return 0
    lax.fori_loop(0, NCH // BLK, blkbody, 0)

    plsc.subcore_barrier()
    pltpu.sync_copy(acc_sh.at[pl.ds(row0, ROWS_PER_TILE)],
                    out_hbm.at[cid, pl.ds(row0, ROWS_PER_TILE)])


def _mm_body(comp_ref, bases_ref, root_ref, x_ref, xall_ref, xroot_ref):
    x = x_ref[...]
    for r in range(R):
        w = comp_ref[r, 0] * bases_ref[0]
        for b in range(1, NB):
            w = w + comp_ref[r, b] * bases_ref[b]
        xall_ref[:, r * H:(r + 1) * H] = jnp.dot(
            x, w, preferred_element_type=jnp.float32)
    xroot_ref[...] = jnp.dot(x, root_ref[...],
                             preferred_element_type=jnp.float32)


_BM = 1024
_mm = pl.pallas_call(
    _mm_body,
    grid=(NPAD // _BM,),
    in_specs=[
        pl.BlockSpec(memory_space=pltpu.SMEM),
        pl.BlockSpec((NB, H, H), lambda i: (0, 0, 0)),
        pl.BlockSpec((H, H), lambda i: (0, 0)),
        pl.BlockSpec((_BM, H), lambda i: (i, 0)),
    ],
    out_specs=[
        pl.BlockSpec((_BM, R * H), lambda i: (i, 0)),
        pl.BlockSpec((_BM, H), lambda i: (i, 0)),
    ],
    out_shape=[
        jax.ShapeDtypeStruct((NPAD, R * H), jnp.float32),
        jax.ShapeDtypeStruct((NPAD, H), jnp.float32),
    ],
)


def _bn_relu_residual(part_ref, xroot_ref, x_ref, bias_ref, bnw_ref, bnb_ref):
    h = part_ref[0] + part_ref[1] + xroot_ref[...] + bias_ref[...]
    rows = lax.broadcasted_iota(jnp.int32, (NPAD, 1), 0)
    m = rows < N
    hm = jnp.where(m, h, 0.0)
    mu = jnp.sum(hm, axis=0, keepdims=True) * (1.0 / N)
    d = h - mu
    var = jnp.sum(jnp.where(m, d * d, 0.0), axis=0, keepdims=True) * (1.0 / N)
    hn = d / jnp.sqrt(var + EPS) * bnw_ref[...] + bnb_ref[...]
    return x_ref[...] + jnp.maximum(hn, 0.0)


def _mmc_body(part_ref, xroot_ref, x_ref, bias_ref, bnw_ref, bnb_ref,
              comp_ref, bases_ref, root_ref, xnew_ref, xall_ref, xroot2_ref):
    xn = _bn_relu_residual(part_ref, xroot_ref, x_ref, bias_ref, bnw_ref,
                           bnb_ref)
    xnew_ref[...] = xn
    for r in range(R):
        w = comp_ref[r, 0] * bases_ref[0]
        for b in range(1, NB):
            w = w + comp_ref[r, b] * bases_ref[b]
        xall_ref[:, r * H:(r + 1) * H] = jnp.dot(
            xn, w, preferred_element_type=jnp.float32)
    xroot2_ref[...] = jnp.dot(xn, root_ref[...],
                              preferred_element_type=jnp.float32)


_mmc = pl.pallas_call(
    _mmc_body,
    in_specs=[
        pl.BlockSpec(memory_space=pltpu.VMEM),
        pl.BlockSpec(memory_space=pltpu.VMEM),
        pl.BlockSpec(memory_space=pltpu.VMEM),
        pl.BlockSpec(memory_space=pltpu.VMEM),
        pl.BlockSpec(memory_space=pltpu.VMEM),
        pl.BlockSpec(memory_space=pltpu.VMEM),
        pl.BlockSpec(memory_space=pltpu.SMEM),
        pl.BlockSpec(memory_space=pltpu.VMEM),
        pl.BlockSpec(memory_space=pltpu.VMEM),
    ],
    out_shape=[
        jax.ShapeDtypeStruct((NPAD, H), jnp.float32),
        jax.ShapeDtypeStruct((NPAD, R * H), jnp.float32),
        jax.ShapeDtypeStruct((NPAD, H), jnp.float32),
    ],
)


def _pool_body(part_ref, xroot_ref, x_ref, bias_ref, bnw_ref, bnb_ref,
               batch_ref, l1m_ref, l1x_ref, l1b_ref, lo_ref, lob_ref,
               t_ref, out_ref, xm_ref, xx_ref):
    x = _bn_relu_residual(part_ref, xroot_ref, x_ref, bias_ref, bnw_ref,
                          bnb_ref)
    b = batch_ref[...]
    for g in range(G):
        m = b == g
        cnt = jnp.sum(jnp.where(m, 1.0, 0.0))
        s = jnp.sum(jnp.where(m, x, 0.0), axis=0)
        xm_ref[g, :] = s / jnp.maximum(cnt, 1.0)
        mx = jnp.max(jnp.where(m, x, -jnp.inf), axis=0)
        xx_ref[g, :] = jnp.where(mx > -1e37, mx, 0.0)
    hidden = jnp.maximum(
        jnp.dot(xm_ref[...], l1m_ref[...], preferred_element_type=jnp.float32)
        + jnp.dot(xx_ref[...], l1x_ref[...], preferred_element_type=jnp.float32)
        + l1b_ref[...], 0.0)
    logits = jnp.dot(hidden, lo_ref[...],
                     preferred_element_type=jnp.float32) + lob_ref[...]
    t = jnp.maximum(t_ref[0, 0], 1e-4)
    out_ref[...] = logits / t


_pool = pl.pallas_call(
    _pool_body,
    in_specs=[
        pl.BlockSpec(memory_space=pltpu.VMEM),
        pl.BlockSpec(memory_space=pltpu.VMEM),
        pl.BlockSpec(memory_space=pltpu.VMEM),
        pl.BlockSpec(memory_space=pltpu.VMEM),
        pl.BlockSpec(memory_space=pltpu.VMEM),
        pl.BlockSpec(memory_space=pltpu.VMEM),
        pl.BlockSpec(memory_space=pltpu.VMEM),
        pl.BlockSpec(memory_space=pltpu.VMEM),
        pl.BlockSpec(memory_space=pltpu.VMEM),
        pl.BlockSpec(memory_space=pltpu.VMEM),
        pl.BlockSpec(memory_space=pltpu.VMEM),
        pl.BlockSpec(memory_space=pltpu.VMEM),
        pl.BlockSpec(memory_space=pltpu.SMEM),
    ],
    out_shape=jax.ShapeDtypeStruct((G, H), jnp.float32),
    scratch_shapes=[
        pltpu.VMEM((G, H), jnp.float32),
        pltpu.VMEM((G, H), jnp.float32),
    ],
)


def kernel(x_ids, edge_index, edge_type, batch, emb, bases, comp, root,
           conv_bias, bn_w, bn_b, lin1_w, lin1_b, lout_w, lout_b,
           temperature):
    src = edge_index[0].astype(jnp.int32)
    dst = edge_index[1].astype(jnp.int32)
    et = edge_type.astype(jnp.int32)
    gidx = src * R + et
    cidx = dst * R + et
    xids_pad = jnp.pad(x_ids.astype(jnp.int32), (0, NPAD - N))
    cidx3 = jnp.pad(cidx, (0, CP - E),
                    constant_values=NRP - 1).reshape(NS, CCH, CHUNK)

    x, w = _prep(xids_pad, cidx3, cidx, emb)

    pad_idx = jnp.arange(EP - E, dtype=jnp.int32)
    gidx3 = jnp.concatenate([gidx, pad_idx % (N * R)]).reshape(NW, NCH, CHUNK)
    dst3 = jnp.concatenate([dst, pad_idx % N]).reshape(NW, 2 * NCH, CHUNK // 2)
    w3 = jnp.concatenate(
        [w, jnp.zeros((EP - E,), jnp.float32)]).reshape(NW, NCH, CHUNK)

    xall, xroot = _mm(comp[0], bases[0], root[0], x)
    for l in range(L - 1):
        part = _edge_agg(xall.reshape(NPAD * R, H), gidx3, dst3, w3)
        x, xall, xroot = _mmc(part, xroot, x, conv_bias[l].reshape(1, H),
                              bn_w[l].reshape(1, H), bn_b[l].reshape(1, H),
                              comp[l + 1], bases[l + 1], root[l + 1])
    part = _edge_agg(xall.reshape(NPAD * R, H), gidx3, dst3, w3)

    batch_pad = jnp.pad(batch.astype(jnp.int32), (0, NPAD - N),
                        constant_values=G).reshape(NPAD, 1)
    l1t = lin1_w.T
    lo_pad = jnp.pad(lout_w.T, ((0, 0), (0, H - 2)))
    lob_pad = jnp.pad(lout_b, (0, H - 2)).reshape(1, H)
    out = _pool(part, xroot, x, conv_bias[L - 1].reshape(1, H),
                bn_w[L - 1].reshape(1, H), bn_b[L - 1].reshape(1, H),
                batch_pad, l1t[:H], l1t[H:], lin1_b.reshape(1, H),
                lo_pad, lob_pad, temperature.reshape(1, 1))
    return out[:, :2]


# speedup vs baseline: 1.1072x; 1.0590x over previous
import functools

import jax
import jax.numpy as jnp
from jax import lax
from jax.experimental import pallas as pl
from jax.experimental.pallas import tpu as pltpu
from jax.experimental.pallas import tpu_sc as plsc

N = 10000
E = 320000
H = 128
R = 4
NB = 4
L = 3
G = 16
EPS = 1e-5

NC = 2
NS = 16
NW = NC * NS

NPAD = 10240
NRP = 40960
CHUNK = 128
EPW = E // NW
ROWS_PER_TILE = NPAD // NS
NCH = 80
BLK = 16
EP = NW * NCH * CHUNK
CCH = 157
CP = NS * CCH * CHUNK

_sc_mesh = plsc.VectorSubcoreMesh(core_axis_name="c", subcore_axis_name="s")
_sc_params = pltpu.CompilerParams(needs_layout_passes=False)


@functools.partial(
    pl.kernel,
    out_type=jax.ShapeDtypeStruct((NPAD, H), jnp.float32),
    mesh=_sc_mesh,
    scratch_types=[
        pltpu.VMEM((CHUNK,), jnp.int32),
        pltpu.VMEM((CHUNK, H), jnp.float32),
        pltpu.SemaphoreType.DMA,
    ],
    compiler_params=_sc_params,
)
def _embed(xids_hbm, emb_hbm, x_hbm, idbuf, rowbuf, sem):
    cid = lax.axis_index("c")
    sid = lax.axis_index("s")
    wid = sid * NC + cid

    nemb = NPAD // CHUNK
    def embody(j, _):
        c = wid + NW * j
        @pl.when(c < nemb)
        def _():
            base = c * CHUNK
            pltpu.sync_copy(xids_hbm.at[pl.ds(base, CHUNK)], idbuf)
            pltpu.async_copy(emb_hbm.at[idbuf], rowbuf, sem).wait()
            pltpu.sync_copy(rowbuf, x_hbm.at[pl.ds(base, CHUNK)])
        return 0
    lax.fori_loop(0, (nemb + NW - 1) // NW, embody, 0)


@functools.partial(
    pl.kernel,
    out_type=jax.ShapeDtypeStruct((E,), jnp.float32),
    mesh=_sc_mesh,
    scratch_types=[
        pltpu.VMEM_SHARED((NRP,), jnp.float32),
        pltpu.VMEM((CHUNK,), jnp.float32),
        pltpu.VMEM((NRP // NS,), jnp.float32),
        pltpu.VMEM((NRP,), jnp.float32),
        pltpu.VMEM((CCH, CHUNK), jnp.int32),
        pltpu.VMEM((EPW,), jnp.int32),
        pltpu.VMEM((EPW,), jnp.float32),
        pltpu.SemaphoreType.DMA,
    ],
    compiler_params=_sc_params,
)
def _weights(cidx3_hbm, cidx_hbm, w_hbm,
             cnt_sh, onesbuf, slbuf, recipbuf, cq2, cqbuf, woutbuf, sem):
    cid = lax.axis_index("c")
    sid = lax.axis_index("s")
    wid = sid * NC + cid

    slice_len = NRP // NS

    def zstore(i, _):
        slbuf[pl.ds(i * 16, 16)] = jnp.zeros((16,), jnp.float32)
        return 0
    lax.fori_loop(0, slice_len // 16, zstore, 0)
    pltpu.sync_copy(slbuf, cnt_sh.at[pl.ds(sid * slice_len, slice_len)])

    def ostore(i, _):
        onesbuf[pl.ds(i * 16, 16)] = jnp.ones((16,), jnp.float32)
        return 0
    lax.fori_loop(0, CHUNK // 16, ostore, 0)

    plsc.subcore_barrier()

    pltpu.sync_copy(cidx3_hbm.at[sid], cq2)
    def cbody(j, _):
        pltpu.sync_copy(onesbuf, cnt_sh.at[cq2.at[j]], add=True)
        return 0
    lax.fori_loop(0, CCH, cbody, 0)

    plsc.subcore_barrier()

    sl = pl.ds(sid * slice_len, slice_len)
    pltpu.sync_copy(cnt_sh.at[sl], slbuf)
    def rbody(i, _):
        v = slbuf[pl.ds(i * 16, 16)]
        slbuf[pl.ds(i * 16, 16)] = 1.0 / jnp.maximum(v, 1.0)
        return 0
    lax.fori_loop(0, slice_len // 16, rbody, 0)
    pltpu.sync_copy(slbuf, cnt_sh.at[sl])

    plsc.subcore_barrier()

    pltpu.sync_copy(cnt_sh, recipbuf)
    wbase = wid * EPW
    pltpu.sync_copy(cidx_hbm.at[pl.ds(wbase, EPW)], cqbuf)
    def wbody(i, _):
        idxv = cqbuf[pl.ds(i * 16, 16)]
        woutbuf[pl.ds(i * 16, 16)] = plsc.load_gather(recipbuf, [idxv])
        return 0
    lax.fori_loop(0, EPW // 16, wbody, 0)
    pltpu.sync_copy(woutbuf, w_hbm.at[pl.ds(wbase, EPW)])


@functools.partial(
    pl.kernel,
    out_type=jax.ShapeDtypeStruct((NC, NPAD, H), jnp.float32),
    mesh=_sc_mesh,
    scratch_types=[
        pltpu.VMEM_SHARED((NPAD, H), jnp.float32),
        pltpu.VMEM((BLK, CHUNK), jnp.int32),
        pltpu.VMEM((2 * BLK, CHUNK // 2), jnp.int32),
        pltpu.VMEM((BLK, CHUNK), jnp.float32),
        pltpu.VMEM((CHUNK, H), jnp.float32),
        pltpu.VMEM((CHUNK, H), jnp.float32),
        pltpu.SemaphoreType.DMA,
        pltpu.SemaphoreType.DMA,
        pltpu.SemaphoreType.DMA,
    ],
    compiler_params=_sc_params,
)
def _edge_agg(xall_hbm, gidx_hbm, dst_hbm, w_hbm, out_hbm,
              acc_sh, gi2, di2, w2, rows0, rows1, sem0, sem1, sem_s):
    cid = lax.axis_index("c")
    sid = lax.axis_index("s")
    wid = sid * NC + cid

    def zrow(i, _):
        def zcol(i16, _):
            rows0[i, pl.ds(i16 * 16, 16)] = jnp.zeros((16,), jnp.float32)
            return 0
        lax.fori_loop(0, H // 16, zcol, 0)
        return 0
    lax.fori_loop(0, CHUNK, zrow, 0)
    row0 = sid * ROWS_PER_TILE
    for p in range(ROWS_PER_TILE // CHUNK):
        pltpu.sync_copy(rows0, acc_sh.at[pl.ds(row0 + p * CHUNK, CHUNK)])

    plsc.subcore_barrier()

    def blkbody(bi, _):
        pltpu.sync_copy(gidx_hbm.at[wid, pl.ds(bi * BLK, BLK)], gi2)
        pltpu.sync_copy(dst_hbm.at[wid, pl.ds(bi * 2 * BLK, 2 * BLK)], di2)
        pltpu.sync_copy(w_hbm.at[wid, pl.ds(bi * BLK, BLK)], w2)
        pltpu.async_copy(xall_hbm.at[gi2.at[0]], rows0, sem0)
        pltpu.async_copy(xall_hbm.at[gi2.at[1]], rows1, sem1)
        def ebody(jj, _):
            for b, rows, sem in ((0, rows0, sem0), (1, rows1, sem1)):
                j = 2 * jj + b
                pltpu.make_async_copy(xall_hbm.at[gi2.at[j]], rows,
                                      sem).wait()
                def scale_half(lo):
                    def sgroup(k16):
                        wg = w2[j, pl.ds(k16 * 16, 16)]
                        for i in range(16):
                            k = k16 * 16 + i
                            sb = jnp.full((16,), wg[i], jnp.float32)
                            for i16 in range(H // 16):
                                rows[k, pl.ds(i16 * 16, 16)] = (
                                    rows[k, pl.ds(i16 * 16, 16)] * sb)
                    plsc.parallel_loop(lo, lo + CHUNK // 32, 1,
                                       unroll=2)(sgroup)
                scale_half(0)
                desc = pltpu.async_copy(rows.at[pl.ds(0, CHUNK // 2)],
                                        acc_sh.at[di2.at[2 * j]], sem_s,
                                        add=True)
                scale_half(CHUNK // 32)
                pltpu.sync_copy(rows.at[pl.ds(CHUNK // 2, CHUNK // 2)],
                                acc_sh.at[di2.at[2 * j + 1]], add=True)
                desc.wait()
                @pl.when(jj < BLK // 2 - 1)
                def _():
                    pltpu.async_copy(xall_hbm.at[gi2.at[j + 2]], rows, sem)
            return 0
        lax.fori_loop(0, BLK // 2, ebody, 0)
        return 0
    lax.fori_loop(0, NCH // BLK, blkbody, 0)

    plsc.subcore_barrier()
    pltpu.sync_copy(acc_sh.at[pl.ds(row0, ROWS_PER_TILE)],
                    out_hbm.at[cid, pl.ds(row0, ROWS_PER_TILE)])


def _mm_body(comp_ref, bases_ref, root_ref, x_ref, xall_ref, xroot_ref):
    x = x_ref[...]
    for r in range(R):
        w = comp_ref[r, 0] * bases_ref[0]
        for b in range(1, NB):
            w = w + comp_ref[r, b] * bases_ref[b]
        xall_ref[:, r * H:(r + 1) * H] = jnp.dot(
            x, w, preferred_element_type=jnp.float32)
    xroot_ref[...] = jnp.dot(x, root_ref[...],
                             preferred_element_type=jnp.float32)


_BM = 1024
_mm = pl.pallas_call(
    _mm_body,
    grid=(NPAD // _BM,),
    in_specs=[
        pl.BlockSpec(memory_space=pltpu.SMEM),
        pl.BlockSpec((NB, H, H), lambda i: (0, 0, 0)),
        pl.BlockSpec((H, H), lambda i: (0, 0)),
        pl.BlockSpec((_BM, H), lambda i: (i, 0)),
    ],
    out_specs=[
        pl.BlockSpec((_BM, R * H), lambda i: (i, 0)),
        pl.BlockSpec((_BM, H), lambda i: (i, 0)),
    ],
    out_shape=[
        jax.ShapeDtypeStruct((NPAD, R * H), jnp.float32),
        jax.ShapeDtypeStruct((NPAD, H), jnp.float32),
    ],
)


def _bn_relu_residual(part_ref, xroot_ref, x_ref, bias_ref, bnw_ref, bnb_ref):
    h = part_ref[0] + part_ref[1] + xroot_ref[...] + bias_ref[...]
    rows = lax.broadcasted_iota(jnp.int32, (NPAD, 1), 0)
    m = rows < N
    hm = jnp.where(m, h, 0.0)
    mu = jnp.sum(hm, axis=0, keepdims=True) * (1.0 / N)
    d = h - mu
    var = jnp.sum(jnp.where(m, d * d, 0.0), axis=0, keepdims=True) * (1.0 / N)
    hn = d / jnp.sqrt(var + EPS) * bnw_ref[...] + bnb_ref[...]
    return x_ref[...] + jnp.maximum(hn, 0.0)


def _mmc_body(part_ref, xroot_ref, x_ref, bias_ref, bnw_ref, bnb_ref,
              comp_ref, bases_ref, root_ref, xnew_ref, xall_ref, xroot2_ref):
    xn = _bn_relu_residual(part_ref, xroot_ref, x_ref, bias_ref, bnw_ref,
                           bnb_ref)
    xnew_ref[...] = xn
    for r in range(R):
        w = comp_ref[r, 0] * bases_ref[0]
        for b in range(1, NB):
            w = w + comp_ref[r, b] * bases_ref[b]
        xall_ref[:, r * H:(r + 1) * H] = jnp.dot(
            xn, w, preferred_element_type=jnp.float32)
    xroot2_ref[...] = jnp.dot(xn, root_ref[...],
                              preferred_element_type=jnp.float32)


_mmc = pl.pallas_call(
    _mmc_body,
    in_specs=[
        pl.BlockSpec(memory_space=pltpu.VMEM),
        pl.BlockSpec(memory_space=pltpu.VMEM),
        pl.BlockSpec(memory_space=pltpu.VMEM),
        pl.BlockSpec(memory_space=pltpu.VMEM),
        pl.BlockSpec(memory_space=pltpu.VMEM),
        pl.BlockSpec(memory_space=pltpu.VMEM),
        pl.BlockSpec(memory_space=pltpu.SMEM),
        pl.BlockSpec(memory_space=pltpu.VMEM),
        pl.BlockSpec(memory_space=pltpu.VMEM),
    ],
    out_shape=[
        jax.ShapeDtypeStruct((NPAD, H), jnp.float32),
        jax.ShapeDtypeStruct((NPAD, R * H), jnp.float32),
        jax.ShapeDtypeStruct((NPAD, H), jnp.float32),
    ],
)


def _pool_body(part_ref, xroot_ref, x_ref, bias_ref, bnw_ref, bnb_ref,
               batch_ref, l1m_ref, l1x_ref, l1b_ref, lo_ref, lob_ref,
               t_ref, out_ref, xm_ref, xx_ref):
    x = _bn_relu_residual(part_ref, xroot_ref, x_ref, bias_ref, bnw_ref,
                          bnb_ref)
    b = batch_ref[...]
    for g in range(G):
        m = b == g
        cnt = jnp.sum(jnp.where(m, 1.0, 0.0))
        s = jnp.sum(jnp.where(m, x, 0.0), axis=0)
        xm_ref[g, :] = s / jnp.maximum(cnt, 1.0)
        mx = jnp.max(jnp.where(m, x, -jnp.inf), axis=0)
        xx_ref[g, :] = jnp.where(mx > -1e37, mx, 0.0)
    hidden = jnp.maximum(
        jnp.dot(xm_ref[...], l1m_ref[...], preferred_element_type=jnp.float32)
        + jnp.dot(xx_ref[...], l1x_ref[...], preferred_element_type=jnp.float32)
        + l1b_ref[...], 0.0)
    logits = jnp.dot(hidden, lo_ref[...],
                     preferred_element_type=jnp.float32) + lob_ref[...]
    t = jnp.maximum(t_ref[0, 0], 1e-4)
    out_ref[...] = logits / t


_pool = pl.pallas_call(
    _pool_body,
    in_specs=[
        pl.BlockSpec(memory_space=pltpu.VMEM),
        pl.BlockSpec(memory_space=pltpu.VMEM),
        pl.BlockSpec(memory_space=pltpu.VMEM),
        pl.BlockSpec(memory_space=pltpu.VMEM),
        pl.BlockSpec(memory_space=pltpu.VMEM),
        pl.BlockSpec(memory_space=pltpu.VMEM),
        pl.BlockSpec(memory_space=pltpu.VMEM),
        pl.BlockSpec(memory_space=pltpu.VMEM),
        pl.BlockSpec(memory_space=pltpu.VMEM),
        pl.BlockSpec(memory_space=pltpu.VMEM),
        pl.BlockSpec(memory_space=pltpu.VMEM),
        pl.BlockSpec(memory_space=pltpu.VMEM),
        pl.BlockSpec(memory_space=pltpu.SMEM),
    ],
    out_shape=jax.ShapeDtypeStruct((G, H), jnp.float32),
    scratch_shapes=[
        pltpu.VMEM((G, H), jnp.float32),
        pltpu.VMEM((G, H), jnp.float32),
    ],
)


def kernel(x_ids, edge_index, edge_type, batch, emb, bases, comp, root,
           conv_bias, bn_w, bn_b, lin1_w, lin1_b, lout_w, lout_b,
           temperature):
    src = edge_index[0].astype(jnp.int32)
    dst = edge_index[1].astype(jnp.int32)
    et = edge_type.astype(jnp.int32)
    gidx = src * R + et
    cidx = dst * R + et
    xids_pad = jnp.pad(x_ids.astype(jnp.int32), (0, NPAD - N))
    cidx3 = jnp.pad(cidx, (0, CP - E),
                    constant_values=NRP - 1).reshape(NS, CCH, CHUNK)

    x = _embed(xids_pad, emb)
    w = _weights(cidx3, cidx)

    pad_idx = jnp.arange(EP - E, dtype=jnp.int32)
    gidx3 = jnp.concatenate([gidx, pad_idx % (N * R)]).reshape(NW, NCH, CHUNK)
    dst3 = jnp.concatenate([dst, pad_idx % N]).reshape(NW, 2 * NCH, CHUNK // 2)
    w3 = jnp.concatenate(
        [w, jnp.zeros((EP - E,), jnp.float32)]).reshape(NW, NCH, CHUNK)

    xall, xroot = _mm(comp[0], bases[0], root[0], x)
    for l in range(L - 1):
        part = _edge_agg(xall.reshape(NPAD * R, H), gidx3, dst3, w3)
        x, xall, xroot = _mmc(part, xroot, x, conv_bias[l].reshape(1, H),
                              bn_w[l].reshape(1, H), bn_b[l].reshape(1, H),
                              comp[l + 1], bases[l + 1], root[l + 1])
    part = _edge_agg(xall.reshape(NPAD * R, H), gidx3, dst3, w3)

    batch_pad = jnp.pad(batch.astype(jnp.int32), (0, NPAD - N),
                        constant_values=G).reshape(NPAD, 1)
    l1t = lin1_w.T
    lo_pad = jnp.pad(lout_w.T, ((0, 0), (0, H - 2)))
    lob_pad = jnp.pad(lout_b, (0, H - 2)).reshape(1, H)
    out = _pool(part, xroot, x, conv_bias[L - 1].reshape(1, H),
                bn_w[L - 1].reshape(1, H), bn_b[L - 1].reshape(1, H),
                batch_pad, l1t[:H], l1t[H:], lin1_b.reshape(1, H),
                lo_pad, lob_pad, temperature.reshape(1, 1))
    return out[:, :2]
